# Initial kernel scaffold; baseline (speedup 1.0000x reference)
import jax
import jax.numpy as jnp
from jax.experimental import pallas as pl


def kernel(xyz_dense, xyz_sparse, feat_sparse, W1, b1, W2, b2):
    raise NotImplementedError("write your pallas kernel here")


# speedup vs baseline: 33.3667x; 33.3667x over previous
import functools

import jax
import jax.numpy as jnp
from jax.experimental import pallas as pl

B, N2, N1, C, O, K = 4, 8192, 2048, 256, 256, 3
TILE = 256


def _fp_body(xd_ref, xs_ref, fs_ref, w1_ref, b1_ref, w2_ref, b2_ref, out_ref):
    xd = xd_ref[0]
    xs = xs_ref[0]
    x2 = jnp.sum(xd * xd, axis=1, keepdims=True)
    y2 = jnp.sum(xs * xs, axis=1)[None, :]
    xy = jax.lax.dot_general(
        xd, xs, (((1,), (1,)), ((), ())), preferred_element_type=jnp.float32
    )
    dist2 = jnp.maximum(x2 + y2 - 2.0 * xy, 1e-12)

    iota = jax.lax.broadcasted_iota(jnp.int32, dist2.shape, 1)
    acc = jnp.zeros(dist2.shape, jnp.float32)
    wsum = jnp.zeros((dist2.shape[0], 1), jnp.float32)
    d = dist2
    for _ in range(K):
        m = jnp.min(d, axis=1, keepdims=True)
        am = jnp.min(jnp.where(d == m, iota, N1), axis=1, keepdims=True)
        sel = iota == am
        w = 1.0 / (jnp.sqrt(m) + 1e-10)
        acc = acc + jnp.where(sel, w, 0.0)
        wsum = wsum + w
        d = jnp.where(sel, jnp.float32(jnp.inf), d)

    feat = jax.lax.dot_general(
        acc, fs_ref[0], (((1,), (0,)), ((), ())), preferred_element_type=jnp.float32
    ) / wsum
    h = jnp.maximum(
        jax.lax.dot_general(
            feat, w1_ref[...], (((1,), (0,)), ((), ())),
            preferred_element_type=jnp.float32,
        ) + b1_ref[...],
        0.0,
    )
    out_ref[0] = jax.lax.dot_general(
        h, w2_ref[...], (((1,), (0,)), ((), ())),
        preferred_element_type=jnp.float32,
    ) + b2_ref[...]


@jax.jit
def kernel(xyz_dense, xyz_sparse, feat_sparse, W1, b1, W2, b2):
    b1r = b1.reshape(1, O)
    b2r = b2.reshape(1, O)
    grid = (B, N2 // TILE)
    return pl.pallas_call(
        _fp_body,
        grid=grid,
        in_specs=[
            pl.BlockSpec((1, TILE, 3), lambda b, t: (b, t, 0)),
            pl.BlockSpec((1, N1, 3), lambda b, t: (b, 0, 0)),
            pl.BlockSpec((1, N1, C), lambda b, t: (b, 0, 0)),
            pl.BlockSpec((C, O), lambda b, t: (0, 0)),
            pl.BlockSpec((1, O), lambda b, t: (0, 0)),
            pl.BlockSpec((O, O), lambda b, t: (0, 0)),
            pl.BlockSpec((1, O), lambda b, t: (0, 0)),
        ],
        out_specs=pl.BlockSpec((1, TILE, O), lambda b, t: (b, t, 0)),
        out_shape=jax.ShapeDtypeStruct((B, N2, O), jnp.float32),
    )(xyz_dense, xyz_sparse, feat_sparse, W1, b1r, W2, b2r)


# speedup vs baseline: 39.8116x; 1.1932x over previous
import jax
import jax.numpy as jnp
from jax.experimental import pallas as pl

B, N2, N1, C, O, K = 4, 8192, 2048, 256, 256, 3
TILE = 512


def _fp_body(xdn2_ref, xs_ref, fs_ref, w1_ref, b1_ref, w2_ref, b2_ref, out_ref):
    xdn2 = xdn2_ref[0]
    xs = xs_ref[0]
    x2 = 0.25 * jnp.sum(xdn2 * xdn2, axis=1, keepdims=True)
    y2 = jnp.sum(xs * xs, axis=1)[None, :]
    xy_n2 = jax.lax.dot_general(
        xdn2, xs, (((1,), (1,)), ((), ())), preferred_element_type=jnp.float32
    )
    dist2 = jnp.maximum((x2 + y2) + xy_n2, 1e-12)

    big = jnp.float32(jnp.inf)
    iotaf = jax.lax.broadcasted_iota(jnp.int32, dist2.shape, 1).astype(jnp.float32)
    nf = jnp.float32(N1)
    acc = jnp.zeros(dist2.shape, jnp.float32)
    wsum = jnp.zeros((dist2.shape[0], 1), jnp.float32)
    d = dist2
    for k in range(K):
        m = jnp.min(d, axis=1, keepdims=True)
        t = jnp.where(d == m, iotaf, nf)
        am = jnp.min(t, axis=1, keepdims=True)
        sel = t == am
        w = jax.lax.rsqrt(m)
        acc = acc + jnp.where(sel, w, 0.0)
        wsum = wsum + w
        if k < K - 1:
            d = jnp.where(sel, big, d)

    feat = jax.lax.dot_general(
        acc.astype(jnp.bfloat16), fs_ref[0], (((1,), (0,)), ((), ())), preferred_element_type=jnp.float32
    ) / wsum
    h = jnp.maximum(
        jax.lax.dot_general(
            feat.astype(jnp.bfloat16), w1_ref[...],
            (((1,), (0,)), ((), ())), preferred_element_type=jnp.float32,
        ) + b1_ref[...],
        0.0,
    )
    out_ref[0] = jax.lax.dot_general(
        h.astype(jnp.bfloat16), w2_ref[...],
        (((1,), (0,)), ((), ())), preferred_element_type=jnp.float32,
    ) + b2_ref[...]


@jax.jit
def kernel(xyz_dense, xyz_sparse, feat_sparse, W1, b1, W2, b2):
    xdn2 = xyz_dense * jnp.float32(-2.0)
    fs_bf = feat_sparse.astype(jnp.bfloat16)
    w1_bf = W1.astype(jnp.bfloat16)
    w2_bf = W2.astype(jnp.bfloat16)
    b1r = b1.reshape(1, O)
    b2r = b2.reshape(1, O)
    grid = (B, N2 // TILE)
    return pl.pallas_call(
        _fp_body,
        grid=grid,
        in_specs=[
            pl.BlockSpec((1, TILE, 3), lambda b, t: (b, t, 0)),
            pl.BlockSpec((1, N1, 3), lambda b, t: (b, 0, 0)),
            pl.BlockSpec((1, N1, C), lambda b, t: (b, 0, 0)),
            pl.BlockSpec((C, O), lambda b, t: (0, 0)),
            pl.BlockSpec((1, O), lambda b, t: (0, 0)),
            pl.BlockSpec((O, O), lambda b, t: (0, 0)),
            pl.BlockSpec((1, O), lambda b, t: (0, 0)),
        ],
        out_specs=pl.BlockSpec((1, TILE, O), lambda b, t: (b, t, 0)),
        out_shape=jax.ShapeDtypeStruct((B, N2, O), jnp.float32),
    )(xdn2, xyz_sparse, fs_bf, w1_bf, b1r, w2_bf, b2r)
